# Initial kernel scaffold; baseline (speedup 1.0000x reference)
import jax
import jax.numpy as jnp
from jax.experimental import pallas as pl


def kernel(mk, mv, qk):
    raise NotImplementedError("write your pallas kernel here")


# speedup vs baseline: 27.5609x; 27.5609x over previous
import functools
import math

import jax
import jax.numpy as jnp
from jax.experimental import pallas as pl
from jax.experimental.pallas import tpu as pltpu

_TOPK = 50
_NB = 128
_NT = 12
_I32_MIN = -(2**31)
_FLIP = 0x7FFFFFFF


def _f32_to_key(a):
    b = jax.lax.bitcast_convert_type(a, jnp.int32)
    return jnp.where(b >= 0, b, jnp.bitwise_xor(b, jnp.int32(_FLIP)))


def _key_to_f32(s):
    b = jnp.where(s >= 0, s, jnp.bitwise_xor(s, jnp.int32(_FLIP)))
    return jax.lax.bitcast_convert_type(b, jnp.float32)


def _body(qi_ref, mk_ref, mv_ref, out_ref, skey_ref, *, m_real, tm, topk):
    nt = mk_ref.shape[1] // tm
    qi = qi_ref[...]
    nb = qi.shape[1]

    def s1(t, carry):
        vmax, cnt0 = carry
        st = pl.multiple_of(t * tm, tm)
        a = jax.lax.dot_general(
            qi, mk_ref[:, pl.ds(st, tm)],
            (((0,), (0,)), ((), ())),
            precision=jax.lax.Precision.DEFAULT,
            preferred_element_type=jnp.float32)
        col = jax.lax.broadcasted_iota(jnp.int32, (nb, tm), 1) + st
        s = jnp.where(col < m_real, _f32_to_key(a), jnp.int32(_I32_MIN))
        skey_ref[:, pl.ds(st, tm)] = s
        vmax = jnp.maximum(vmax, jnp.max(s, axis=1, keepdims=True))
        cnt0 = cnt0 + jnp.sum((s >= 0).astype(jnp.int32), axis=1,
                              keepdims=True)
        return vmax, cnt0

    vmax, cnt0 = jax.lax.fori_loop(
        0, nt, s1,
        (jnp.full((nb, 1), _I32_MIN, jnp.int32),
         jnp.zeros((nb, 1), jnp.int32)))

    lo0 = jnp.where(cnt0 >= topk,
                    jnp.zeros((nb, 1), jnp.int32),
                    jnp.full((nb, 1), _I32_MIN, jnp.int32))

    def s2(i, lo):
        bit = 30 - i
        cand = jnp.bitwise_or(lo, jnp.left_shift(jnp.int32(1), bit))

        def ctile(t, c):
            st = pl.multiple_of(t * tm, tm)
            s = skey_ref[:, pl.ds(st, tm)]
            return c + jnp.sum((s >= cand).astype(jnp.int32), axis=1,
                               keepdims=True)

        cnt = jax.lax.fori_loop(0, nt, ctile,
                                jnp.zeros((nb, 1), jnp.int32))
        return jnp.where(cnt >= topk, cand, lo)

    thr = jax.lax.fori_loop(0, 31, s2, lo0)

    vmax_f = _key_to_f32(vmax)
    cv = mv_ref.shape[0]

    def s3(t, carry):
        ssum, mem = carry
        st = pl.multiple_of(t * tm, tm)
        s = skey_ref[:, pl.ds(st, tm)]
        af = _key_to_f32(s)
        p = jnp.where(s >= thr, jnp.exp(af - vmax_f), 0.0)
        ssum = ssum + jnp.sum(p, axis=1, keepdims=True)
        mem = mem + jax.lax.dot_general(
            p, mv_ref[:, pl.ds(st, tm)],
            (((1,), (1,)), ((), ())),
            precision=jax.lax.Precision.DEFAULT,
            preferred_element_type=jnp.float32)
        return ssum, mem

    ssum, mem = jax.lax.fori_loop(
        0, nt, s3,
        (jnp.zeros((nb, 1), jnp.float32),
         jnp.zeros((nb, cv), jnp.float32)))
    out_ref[...] = mem / ssum


def kernel(mk, mv, qk):
    B, CK, T, H, W = mk.shape
    CV = mv.shape[1]
    M = T * H * W
    N = H * W
    grain = 128 * _NT
    mp = ((M + grain - 1) // grain) * grain
    tm = mp // _NT
    npad = ((N + _NB - 1) // _NB) * _NB

    mkf = jnp.pad(mk.reshape(CK, M), ((0, 0), (0, mp - M)))
    mvf = jnp.pad(mv.reshape(CV, M), ((0, 0), (0, mp - M)))
    qi = qk.reshape(CK, N) * (1.0 / math.sqrt(CK))
    qi_p = jnp.pad(qi, ((0, 0), (0, npad - N)))

    out = pl.pallas_call(
        functools.partial(_body, m_real=M, tm=tm, topk=_TOPK),
        grid=(npad // _NB,),
        in_specs=[
            pl.BlockSpec((CK, _NB), lambda n: (0, n)),
            pl.BlockSpec((CK, mp), lambda n: (0, 0)),
            pl.BlockSpec((CV, mp), lambda n: (0, 0)),
        ],
        out_specs=pl.BlockSpec((_NB, CV), lambda n: (n, 0)),
        out_shape=jax.ShapeDtypeStruct((npad, CV), jnp.float32),
        scratch_shapes=[pltpu.VMEM((_NB, mp), jnp.int32)],
    )(qi_p, mkf, mvf)

    return out[:N, :].T.reshape(B, CV, H, W)


# speedup vs baseline: 34.0794x; 1.2365x over previous
import functools
import math

import jax
import jax.numpy as jnp
from jax.experimental import pallas as pl
from jax.experimental.pallas import tpu as pltpu

_TOPK = 50
_NB = 128
_NT = 12
_I32_MIN = -(2**31)
_FLIP = 0x7FFFFFFF


def _f32_to_key(a):
    b = jax.lax.bitcast_convert_type(a, jnp.int32)
    return jnp.where(b >= 0, b, jnp.bitwise_xor(b, jnp.int32(_FLIP)))


def _key_to_f32(s):
    b = jnp.where(s >= 0, s, jnp.bitwise_xor(s, jnp.int32(_FLIP)))
    return jax.lax.bitcast_convert_type(b, jnp.float32)


def _body(qi_ref, mk_ref, mv_ref, out_ref, skey_ref, *, m_real, n_real, tm,
          topk):
    nt = mk_ref.shape[1] // tm
    mp = mk_ref.shape[1]
    qi = qi_ref[...]
    nb = qi.shape[1]

    def s1(t, carry):
        vmax, cnt0 = carry
        st = pl.multiple_of(t * tm, tm)
        a = jax.lax.dot_general(
            qi, mk_ref[:, pl.ds(st, tm)],
            (((0,), (0,)), ((), ())),
            precision=jax.lax.Precision.DEFAULT,
            preferred_element_type=jnp.float32)
        col = jax.lax.broadcasted_iota(jnp.int32, (nb, tm), 1) + st
        s = jnp.where(col < m_real, _f32_to_key(a), jnp.int32(_I32_MIN))
        skey_ref[:, pl.ds(st, tm)] = s
        vmax = jnp.maximum(vmax, jnp.max(s, axis=1, keepdims=True))
        cnt0 = cnt0 + jnp.sum((s >= 0).astype(jnp.int32), axis=1,
                              keepdims=True)
        return vmax, cnt0

    vmax, cnt0 = jax.lax.fori_loop(
        0, nt, s1,
        (jnp.full((nb, 1), _I32_MIN, jnp.int32),
         jnp.zeros((nb, 1), jnp.int32)))

    lo0 = jnp.where(cnt0 >= topk,
                    jnp.zeros((nb, 1), jnp.int32),
                    jnp.full((nb, 1), _I32_MIN, jnp.int32))
    cl0 = jnp.where(cnt0 >= topk, cnt0, jnp.full((nb, 1), mp, jnp.int32))
    valid = (jax.lax.broadcasted_iota(jnp.int32, (nb, 1), 0)
             + pl.program_id(0) * nb) < n_real

    def s2_cond(carry):
        i, _, cl = carry
        return jnp.logical_and(i < 31,
                               jnp.any(jnp.logical_and(valid, cl != topk)))

    def s2_body(carry):
        i, lo, cl = carry
        bit = 30 - i
        cand = jnp.bitwise_or(lo, jnp.left_shift(jnp.int32(1), bit))

        def ctile(t, c):
            st = pl.multiple_of(t * tm, tm)
            s = skey_ref[:, pl.ds(st, tm)]
            return c + jnp.sum((s >= cand).astype(jnp.int32), axis=1,
                               keepdims=True)

        cnt = jax.lax.fori_loop(0, nt, ctile,
                                jnp.zeros((nb, 1), jnp.int32))
        take = cnt >= topk
        return (i + 1, jnp.where(take, cand, lo), jnp.where(take, cnt, cl))

    _, thr, _ = jax.lax.while_loop(
        s2_cond, s2_body, (jnp.int32(0), lo0, cl0))

    vmax_f = _key_to_f32(vmax)
    cv = mv_ref.shape[0]

    def s3(t, carry):
        ssum, mem = carry
        st = pl.multiple_of(t * tm, tm)
        s = skey_ref[:, pl.ds(st, tm)]
        af = _key_to_f32(s)
        p = jnp.where(s >= thr, jnp.exp(af - vmax_f), 0.0)
        ssum = ssum + jnp.sum(p, axis=1, keepdims=True)
        mem = mem + jax.lax.dot_general(
            p, mv_ref[:, pl.ds(st, tm)],
            (((1,), (1,)), ((), ())),
            precision=jax.lax.Precision.DEFAULT,
            preferred_element_type=jnp.float32)
        return ssum, mem

    ssum, mem = jax.lax.fori_loop(
        0, nt, s3,
        (jnp.zeros((nb, 1), jnp.float32),
         jnp.zeros((nb, cv), jnp.float32)))
    out_ref[...] = mem / ssum


def kernel(mk, mv, qk):
    B, CK, T, H, W = mk.shape
    CV = mv.shape[1]
    M = T * H * W
    N = H * W
    grain = 128 * _NT
    mp = ((M + grain - 1) // grain) * grain
    tm = mp // _NT
    npad = ((N + _NB - 1) // _NB) * _NB

    mkf = jnp.pad(mk.reshape(CK, M), ((0, 0), (0, mp - M)))
    mvf = jnp.pad(mv.reshape(CV, M), ((0, 0), (0, mp - M)))
    qi = qk.reshape(CK, N) * (1.0 / math.sqrt(CK))
    qi_p = jnp.pad(qi, ((0, 0), (0, npad - N)))

    out = pl.pallas_call(
        functools.partial(_body, m_real=M, n_real=N, tm=tm, topk=_TOPK),
        grid=(npad // _NB,),
        in_specs=[
            pl.BlockSpec((CK, _NB), lambda n: (0, n)),
            pl.BlockSpec((CK, mp), lambda n: (0, 0)),
            pl.BlockSpec((CV, mp), lambda n: (0, 0)),
        ],
        out_specs=pl.BlockSpec((_NB, CV), lambda n: (n, 0)),
        out_shape=jax.ShapeDtypeStruct((npad, CV), jnp.float32),
        scratch_shapes=[pltpu.VMEM((_NB, mp), jnp.int32)],
    )(qi_p, mkf, mvf)

    return out[:N, :].T.reshape(B, CV, H, W)
